# baseline (speedup 1.0000x reference)
import jax
import jax.numpy as jnp
from jax.experimental import pallas as pl
from jax.experimental.pallas import tpu as pltpu

N_EXPERTS = 64
K = 8
BLOCK_ROWS = 1024


def _router_kernel(x1_ref, x2_ref, w_ref, b_ref, out_ref, idx_ref):
    logits = (
        jnp.dot(x1_ref[...], w_ref[0], preferred_element_type=jnp.float32)
        + jnp.dot(x2_ref[...], w_ref[1], preferred_element_type=jnp.float32)
        + b_ref[...]
    )
    col = jax.lax.broadcasted_iota(jnp.int32, logits.shape, 1)
    raw = jax.lax.bitcast_convert_type(logits, jnp.int32)
    key = jnp.where(raw < 0, raw ^ jnp.int32(0x7FFFFFFF), raw)
    keys = (key & jnp.int32(~63)) | (jnp.int32(N_EXPERTS - 1) - col)
    kt = keys.T
    work = kt
    idx_rows = []
    m0 = None
    for _ in range(K):
        m = jnp.max(work, axis=0, keepdims=True)
        if m0 is None:
            m0 = m
        idx_rows.append(jnp.int32(N_EXPERTS - 1) - (m & jnp.int32(63)))
        work = jnp.where(work == m, jnp.int32(-(2**31)), work)
    sel = work == jnp.int32(-(2**31))
    vt = jax.lax.bitcast_convert_type(
        jnp.where(kt < 0, kt ^ jnp.int32(0x7FFFFFFF), kt), jnp.float32
    )
    vmax = jax.lax.bitcast_convert_type(
        jnp.where(m0 < 0, m0 ^ jnp.int32(0x7FFFFFFF), m0), jnp.float32
    )
    e = jnp.where(sel, jnp.exp(vt - vmax), 0.0)
    out_ref[...] = (e / jnp.sum(e, axis=0, keepdims=True)).T
    idx_ref[...] = jnp.concatenate(idx_rows, axis=0).T


@jax.jit
def kernel(x, W, b):
    n, d = x.shape
    h = d // 2
    grid = (n // BLOCK_ROWS,)
    out, idx = pl.pallas_call(
        _router_kernel,
        grid=grid,
        in_specs=[
            pl.BlockSpec((BLOCK_ROWS, h), lambda i: (i, 0)),
            pl.BlockSpec((BLOCK_ROWS, h), lambda i: (i, 1)),
            pl.BlockSpec((2, h, N_EXPERTS), lambda i: (0, 0, 0)),
            pl.BlockSpec((1, N_EXPERTS), lambda i: (0, 0)),
        ],
        out_specs=[
            pl.BlockSpec((BLOCK_ROWS, N_EXPERTS), lambda i: (i, 0)),
            pl.BlockSpec((BLOCK_ROWS, K), lambda i: (i, 0)),
        ],
        out_shape=[
            jax.ShapeDtypeStruct((n, N_EXPERTS), jnp.float32),
            jax.ShapeDtypeStruct((n, K), jnp.int32),
        ],
        compiler_params=pltpu.CompilerParams(
            dimension_semantics=("parallel",),
        ),
    )(x, x, W.reshape(2, h, N_EXPERTS), b.reshape(1, N_EXPERTS))
    return (out, idx)


# speedup vs baseline: 1.0092x; 1.0092x over previous
import jax
import jax.numpy as jnp
from jax.experimental import pallas as pl
from jax.experimental.pallas import tpu as pltpu

N_EXPERTS = 64
K = 8
BLOCK_ROWS = 1024


def _router_kernel(x_ref, w_ref, b_ref, out_ref, idx_ref):
    logits = (
        jnp.dot(x_ref[...], w_ref[...], preferred_element_type=jnp.float32)
        + b_ref[...]
    )
    lt = logits.T
    row = jax.lax.broadcasted_iota(jnp.int32, lt.shape, 0)
    work = lt
    idx_rows = []
    m0 = None
    for _ in range(K):
        m = jnp.max(work, axis=0, keepdims=True)
        if m0 is None:
            m0 = m
        amin = jnp.min(
            jnp.where(work == m, row, jnp.int32(N_EXPERTS)), axis=0, keepdims=True
        )
        idx_rows.append(amin)
        work = jnp.where(row == amin, -jnp.inf, work)
    sel = work == -jnp.inf
    e = jnp.where(sel, jnp.exp(lt - m0), 0.0)
    out_ref[...] = (e / jnp.sum(e, axis=0, keepdims=True)).T
    idx_ref[...] = jnp.concatenate(idx_rows, axis=0).T


@jax.jit
def kernel(x, W, b):
    n, d = x.shape
    grid = (n // BLOCK_ROWS,)
    out, idx = pl.pallas_call(
        _router_kernel,
        grid=grid,
        in_specs=[
            pl.BlockSpec((BLOCK_ROWS, d), lambda i: (i, 0)),
            pl.BlockSpec((d, N_EXPERTS), lambda i: (0, 0)),
            pl.BlockSpec((1, N_EXPERTS), lambda i: (0, 0)),
        ],
        out_specs=[
            pl.BlockSpec((BLOCK_ROWS, N_EXPERTS), lambda i: (i, 0)),
            pl.BlockSpec((BLOCK_ROWS, K), lambda i: (i, 0)),
        ],
        out_shape=[
            jax.ShapeDtypeStruct((n, N_EXPERTS), jnp.float32),
            jax.ShapeDtypeStruct((n, K), jnp.int32),
        ],
        compiler_params=pltpu.CompilerParams(
            dimension_semantics=("parallel",),
        ),
    )(x, W, b.reshape(1, N_EXPERTS))
    return (out, idx)
